# baseline (speedup 1.0000x reference)
import jax
import jax.numpy as jnp
from jax.experimental import pallas as pl
from jax.experimental.pallas import tpu as pltpu

T = 2048
D = 1024
DFF = 2048
E = 8
K = 2
BT = 256
BF = 256
NT = (T * K) // BT
NJ = DFF // BF
NS = T * K


def _meta_kernel(rt0_ref, rt1_ref, pos0_ref, pos1_ref, offs_ref):
    rt0 = rt0_ref[...]
    rt1 = rt1_ref[...]
    r128 = jax.lax.broadcasted_iota(jnp.int32, (128, 128), 0)
    c128 = jax.lax.broadcasted_iota(jnp.int32, (128, 128), 1)
    su128 = (r128 < c128).astype(jnp.float32)
    r16 = jax.lax.broadcasted_iota(jnp.int32, (16, 16), 0)
    c16 = jax.lax.broadcasted_iota(jnp.int32, (16, 16), 1)
    sl16 = (r16 > c16).astype(jnp.float32)

    def ranks(m):
        pin = jax.lax.dot_general(m, su128, (((1,), (0,)), ((), ())),
                                  preferred_element_type=jnp.float32)
        rsum = jnp.sum(m, axis=1, keepdims=True)
        rpre = jax.lax.dot_general(sl16, rsum, (((1,), (0,)), ((), ())),
                                   preferred_element_type=jnp.float32)
        return pin + rpre, jnp.sum(rsum)

    pos0 = jnp.zeros((16, 128), jnp.float32)
    pos1 = jnp.zeros((16, 128), jnp.float32)
    off = 0.0
    off_list = []
    for e in range(E):
        m0 = (rt0 == e).astype(jnp.float32)
        m1 = (rt1 == e).astype(jnp.float32)
        rank0, cnt0 = ranks(m0)
        rank1, cnt1 = ranks(m1)
        off_list.append(off)
        pos0 = pos0 + m0 * (off + rank0)
        pos1 = pos1 + m1 * (off + cnt0 + rank1)
        off = off + cnt0 + cnt1
    off_list.append(off)
    pos0_ref[...] = pos0.astype(jnp.int32)
    pos1_ref[...] = pos1.astype(jnp.int32)
    lane = jax.lax.broadcasted_iota(jnp.int32, (1, 16), 1)
    offs = jnp.zeros((1, 16), jnp.float32)
    for idx, v in enumerate(off_list):
        offs = offs + jnp.where(lane == idx, v, 0.0)
    offs_ref[...] = offs.astype(jnp.int32)


def _grouped_kernel(of_ref,
                    posr_ref, posc_ref, rw_ref, x_ref,
                    wg_ref, wu_ref, w2_ref, out_ref,
                    xb_ref, xs_ref, ys_ref):
    j = pl.program_id(0)
    e = pl.program_id(1)

    @pl.when(jnp.logical_and(j == 0, e == 0))
    def _prologue():
        xb_ref[...] = x_ref[...].astype(jnp.bfloat16)
        ys_ref[...] = jnp.zeros_like(ys_ref)
        for q in range(NT):
            rr = jax.lax.broadcasted_iota(jnp.int32, (BT, 1), 0) + q * BT
            p = jnp.logical_or(posr_ref[0:1, :] == rr,
                               posr_ref[1:2, :] == rr).astype(jnp.bfloat16)
            xs_ref[q * BT:(q + 1) * BT, :] = jax.lax.dot_general(
                p, xb_ref[...], (((1,), (0,)), ((), ())),
                preferred_element_type=jnp.float32).astype(jnp.bfloat16)

    lo = of_ref[e]
    hi = of_ref[e + 1]
    wg = wg_ref[0].astype(jnp.bfloat16)
    wu = wu_ref[0].astype(jnp.bfloat16)
    w2b = w2_ref[0].astype(jnp.bfloat16)

    def _tile(q, carry):
        xsl = xs_ref[pl.ds(q * BT, BT), :]
        gate = jax.lax.dot_general(xsl, wg, (((1,), (1,)), ((), ())),
                                   preferred_element_type=jnp.float32)
        up = jax.lax.dot_general(xsl, wu, (((1,), (1,)), ((), ())),
                                 preferred_element_type=jnp.float32)
        h = gate * jax.lax.logistic(gate) * up
        prow = jax.lax.broadcasted_iota(jnp.int32, (BT, 1), 0) + q * BT
        inside = jnp.logical_and(prow >= lo, prow < hi)
        h = jnp.where(inside, h, 0.0)
        y = jax.lax.dot_general(h.astype(jnp.bfloat16), w2b,
                                (((1,), (1,)), ((), ())),
                                preferred_element_type=jnp.float32)
        ys_ref[pl.ds(q * BT, BT), :] += y
        return carry

    jax.lax.fori_loop(lo // BT, (hi + BT - 1) // BT, _tile, 0)

    @pl.when(jnp.logical_and(j == NJ - 1, e == E - 1))
    def _epilogue():
        for q in range(NT):
            rrow = jax.lax.broadcasted_iota(jnp.int32, (1, BT), 1) + q * BT
            pt = (jnp.where(posc_ref[:, 0:1] == rrow, rw_ref[:, 0:1], 0.0) +
                  jnp.where(posc_ref[:, 1:2] == rrow, rw_ref[:, 1:2], 0.0))
            contrib = jax.lax.dot_general(
                pt, ys_ref[q * BT:(q + 1) * BT, :], (((1,), (0,)), ((), ())),
                preferred_element_type=jnp.float32)
            if q == 0:
                out_ref[...] = contrib
            else:
                out_ref[...] += contrib


def kernel(hidden_states, expert_routing_table, router_weights, w13, w2):
    rt = expert_routing_table.astype(jnp.int32)
    rt0 = rt[:, 0].reshape(16, 128)
    rt1 = rt[:, 1].reshape(16, 128)
    pos0, pos1, offs = pl.pallas_call(
        _meta_kernel,
        out_shape=(jax.ShapeDtypeStruct((16, 128), jnp.int32),
                   jax.ShapeDtypeStruct((16, 128), jnp.int32),
                   jax.ShapeDtypeStruct((1, 16), jnp.int32)),
    )(rt0, rt1)
    offs9 = offs[0, :E + 1]

    posr = jnp.concatenate([pos0.reshape(1, T), pos1.reshape(1, T)], axis=0)
    posc = jnp.concatenate([pos0.reshape(T, 1), pos1.reshape(T, 1)], axis=1)

    grid = (NJ, E)
    grid_spec = pltpu.PrefetchScalarGridSpec(
        num_scalar_prefetch=1,
        grid=grid,
        in_specs=[
            pl.BlockSpec((K, T), lambda j, e, OF: (0, 0)),
            pl.BlockSpec((T, K), lambda j, e, OF: (0, 0)),
            pl.BlockSpec((T, K), lambda j, e, OF: (0, 0)),
            pl.BlockSpec((T, D), lambda j, e, OF: (0, 0)),
            pl.BlockSpec((1, BF, D), lambda j, e, OF: (e, j, 0)),
            pl.BlockSpec((1, BF, D), lambda j, e, OF: (e, NJ + j, 0)),
            pl.BlockSpec((1, D, BF), lambda j, e, OF: (e, 0, j)),
        ],
        out_specs=pl.BlockSpec((T, D), lambda j, e, OF: (0, 0)),
        scratch_shapes=[
            pltpu.VMEM((T, D), jnp.bfloat16),
            pltpu.VMEM((NS, D), jnp.bfloat16),
            pltpu.VMEM((NS, D), jnp.float32),
        ],
    )
    out = pl.pallas_call(
        _grouped_kernel,
        grid_spec=grid_spec,
        out_shape=jax.ShapeDtypeStruct((T, D), jnp.float32),
    )(offs9, posr, posc, router_weights, hidden_states, w13, w13, w2)
    return out


# speedup vs baseline: 1.0054x; 1.0054x over previous
import jax
import jax.numpy as jnp
from jax.experimental import pallas as pl
from jax.experimental.pallas import tpu as pltpu

T = 2048
D = 1024
DFF = 2048
E = 8
K = 2
BT = 256
BF = 256
NT = (T * K) // BT
NJ = DFF // BF
NS = T * K


def _meta_kernel(rt0_ref, rt1_ref, pos0_ref, pos1_ref, offs_ref):
    rt0 = rt0_ref[...]
    rt1 = rt1_ref[...]
    r128 = jax.lax.broadcasted_iota(jnp.int32, (128, 128), 0)
    c128 = jax.lax.broadcasted_iota(jnp.int32, (128, 128), 1)
    su128 = (r128 < c128).astype(jnp.float32)
    r16 = jax.lax.broadcasted_iota(jnp.int32, (16, 16), 0)
    c16 = jax.lax.broadcasted_iota(jnp.int32, (16, 16), 1)
    sl16 = (r16 > c16).astype(jnp.float32)

    def ranks(m):
        pin = jax.lax.dot_general(m, su128, (((1,), (0,)), ((), ())),
                                  preferred_element_type=jnp.float32)
        rsum = jnp.sum(m, axis=1, keepdims=True)
        rpre = jax.lax.dot_general(sl16, rsum, (((1,), (0,)), ((), ())),
                                   preferred_element_type=jnp.float32)
        return pin + rpre, jnp.sum(rsum)

    pos0 = jnp.zeros((16, 128), jnp.float32)
    pos1 = jnp.zeros((16, 128), jnp.float32)
    off = 0.0
    off_list = []
    for e in range(E):
        m0 = (rt0 == e).astype(jnp.float32)
        m1 = (rt1 == e).astype(jnp.float32)
        rank0, cnt0 = ranks(m0)
        rank1, cnt1 = ranks(m1)
        off_list.append(off)
        pos0 = pos0 + m0 * (off + rank0)
        pos1 = pos1 + m1 * (off + cnt0 + rank1)
        off = off + cnt0 + cnt1
    off_list.append(off)
    pos0_ref[...] = pos0.astype(jnp.int32)
    pos1_ref[...] = pos1.astype(jnp.int32)
    lane = jax.lax.broadcasted_iota(jnp.int32, (1, 16), 1)
    offs = jnp.zeros((1, 16), jnp.float32)
    for idx, v in enumerate(off_list):
        offs = offs + jnp.where(lane == idx, v, 0.0)
    offs_ref[...] = offs.astype(jnp.int32)


def _grouped_kernel(of_ref,
                    posr_ref, posc_ref, rw_ref, x_ref,
                    wg_ref, wu_ref, w2_ref, out_ref,
                    xb_ref, xs_ref, ys_ref):
    j = pl.program_id(0)
    e = pl.program_id(1)

    @pl.when(jnp.logical_and(j == 0, e == 0))
    def _prologue():
        xb_ref[...] = x_ref[...].astype(jnp.bfloat16)
        ys_ref[...] = jnp.zeros_like(ys_ref)
        for q in range(NT):
            rr = jax.lax.broadcasted_iota(jnp.int32, (BT, 1), 0) + q * BT
            p = jnp.logical_or(posr_ref[0:1, :] == rr,
                               posr_ref[1:2, :] == rr).astype(jnp.bfloat16)
            xs_ref[q * BT:(q + 1) * BT, :] = jax.lax.dot_general(
                p, xb_ref[...], (((1,), (0,)), ((), ())),
                preferred_element_type=jnp.float32).astype(jnp.bfloat16)

    lo = of_ref[e]
    hi = of_ref[e + 1]

    def _tile(q, carry):
        wg = wg_ref[0].astype(jnp.bfloat16)
        wu = wu_ref[0].astype(jnp.bfloat16)
        w2b = w2_ref[0].astype(jnp.bfloat16)
        xsl = xs_ref[pl.ds(q * BT, BT), :]
        gate = jax.lax.dot_general(xsl, wg, (((1,), (1,)), ((), ())),
                                   preferred_element_type=jnp.float32)
        up = jax.lax.dot_general(xsl, wu, (((1,), (1,)), ((), ())),
                                 preferred_element_type=jnp.float32)
        h = gate * jax.lax.logistic(gate) * up
        prow = jax.lax.broadcasted_iota(jnp.int32, (BT, 1), 0) + q * BT
        inside = jnp.logical_and(prow >= lo, prow < hi)
        h = jnp.where(inside, h, 0.0)
        y = jax.lax.dot_general(h.astype(jnp.bfloat16), w2b,
                                (((1,), (1,)), ((), ())),
                                preferred_element_type=jnp.float32)
        ys_ref[pl.ds(q * BT, BT), :] += y
        return carry

    jax.lax.fori_loop(lo // BT, (hi + BT - 1) // BT, _tile, 0)

    @pl.when(jnp.logical_and(j == NJ - 1, e == E - 1))
    def _epilogue():
        for q in range(NT):
            rrow = jax.lax.broadcasted_iota(jnp.int32, (1, BT), 1) + q * BT
            pt = (jnp.where(posc_ref[:, 0:1] == rrow, rw_ref[:, 0:1], 0.0) +
                  jnp.where(posc_ref[:, 1:2] == rrow, rw_ref[:, 1:2], 0.0))
            contrib = jax.lax.dot_general(
                pt, ys_ref[q * BT:(q + 1) * BT, :], (((1,), (0,)), ((), ())),
                preferred_element_type=jnp.float32)
            if q == 0:
                out_ref[...] = contrib
            else:
                out_ref[...] += contrib


def kernel(hidden_states, expert_routing_table, router_weights, w13, w2):
    rt = expert_routing_table.astype(jnp.int32)
    rt0 = rt[:, 0].reshape(16, 128)
    rt1 = rt[:, 1].reshape(16, 128)
    pos0, pos1, offs = pl.pallas_call(
        _meta_kernel,
        out_shape=(jax.ShapeDtypeStruct((16, 128), jnp.int32),
                   jax.ShapeDtypeStruct((16, 128), jnp.int32),
                   jax.ShapeDtypeStruct((1, 16), jnp.int32)),
    )(rt0, rt1)
    offs9 = offs[0, :E + 1]

    posr = jnp.concatenate([pos0.reshape(1, T), pos1.reshape(1, T)], axis=0)
    posc = jnp.concatenate([pos0.reshape(T, 1), pos1.reshape(T, 1)], axis=1)

    grid = (NJ, E)
    grid_spec = pltpu.PrefetchScalarGridSpec(
        num_scalar_prefetch=1,
        grid=grid,
        in_specs=[
            pl.BlockSpec((K, T), lambda j, e, OF: (0, 0)),
            pl.BlockSpec((T, K), lambda j, e, OF: (0, 0)),
            pl.BlockSpec((T, K), lambda j, e, OF: (0, 0)),
            pl.BlockSpec((T, D), lambda j, e, OF: (0, 0)),
            pl.BlockSpec((1, BF, D), lambda j, e, OF: (e, j, 0)),
            pl.BlockSpec((1, BF, D), lambda j, e, OF: (e, NJ + j, 0)),
            pl.BlockSpec((1, D, BF), lambda j, e, OF: (e, 0, j)),
        ],
        out_specs=pl.BlockSpec((T, D), lambda j, e, OF: (0, 0)),
        scratch_shapes=[
            pltpu.VMEM((T, D), jnp.bfloat16),
            pltpu.VMEM((NS, D), jnp.bfloat16),
            pltpu.VMEM((NS, D), jnp.float32),
        ],
    )
    out = pl.pallas_call(
        _grouped_kernel,
        grid_spec=grid_spec,
        out_shape=jax.ShapeDtypeStruct((T, D), jnp.float32),
    )(offs9, posr, posc, router_weights, hidden_states, w13, w13, w2)
    return out


# speedup vs baseline: 7.3267x; 7.2874x over previous
import jax
import jax.numpy as jnp
from jax.experimental import pallas as pl
from jax.experimental.pallas import tpu as pltpu

T = 2048
D = 1024
DFF = 2048
E = 8
K = 2
BF = 512


def _moe_dense_kernel(rt_ref, rw_ref, x_ref, w13g_ref, w13u_ref, w2_ref,
                      out_ref, xb_ref):
    e = pl.program_id(0)
    j = pl.program_id(1)

    @pl.when(jnp.logical_and(e == 0, j == 0))
    def _init():
        xb_ref[...] = x_ref[...].astype(jnp.bfloat16)
        out_ref[...] = jnp.zeros_like(out_ref)

    x = xb_ref[...]
    wg = w13g_ref[0].astype(jnp.bfloat16)
    wu = w13u_ref[0].astype(jnp.bfloat16)
    gate = jax.lax.dot_general(x, wg, (((1,), (1,)), ((), ())),
                               preferred_element_type=jnp.float32)
    up = jax.lax.dot_general(x, wu, (((1,), (1,)), ((), ())),
                             preferred_element_type=jnp.float32)
    h = (gate * jax.lax.logistic(gate)) * up
    w2b = w2_ref[0].astype(jnp.bfloat16)
    y = jax.lax.dot_general(h.astype(jnp.bfloat16), w2b,
                            (((1,), (1,)), ((), ())),
                            preferred_element_type=jnp.float32)

    rt = rt_ref[...]
    rw = rw_ref[...]
    scale = jnp.sum(jnp.where(rt == e, rw, 0.0), axis=1, keepdims=True)

    out_ref[...] += scale * y


def kernel(hidden_states, expert_routing_table, router_weights, w13, w2):
    rt = expert_routing_table.astype(jnp.int32)
    grid = (E, DFF // BF)
    out = pl.pallas_call(
        _moe_dense_kernel,
        grid=grid,
        in_specs=[
            pl.BlockSpec((T, K), lambda e, j: (0, 0)),
            pl.BlockSpec((T, K), lambda e, j: (0, 0)),
            pl.BlockSpec((T, D), lambda e, j: (0, 0)),
            pl.BlockSpec((1, BF, D), lambda e, j: (e, j, 0)),
            pl.BlockSpec((1, BF, D), lambda e, j: (e, DFF // BF + j, 0)),
            pl.BlockSpec((1, D, BF), lambda e, j: (e, 0, j)),
        ],
        out_specs=pl.BlockSpec((T, D), lambda e, j: (0, 0)),
        out_shape=jax.ShapeDtypeStruct((T, D), jnp.float32),
        scratch_shapes=[pltpu.VMEM((T, D), jnp.bfloat16)],
    )(rt, router_weights, hidden_states, w13, w13, w2)
    return out
